# baseline (speedup 1.0000x reference)
import functools

import jax
import jax.numpy as jnp
from jax import lax
from jax.experimental import pallas as pl
from jax.experimental.pallas import tpu as pltpu
from jax.experimental.pallas import tpu_sc as plsc

_UNITS = 100000
_NEG = 1024
_DIM = 64
_BATCH = 16384

_NC = 2
_NS = 16
_NW = _NC * _NS
_BPW = _BATCH // _NW
_SPW = _NEG // _NW
_CHUNK = 128
_NCHUNK = _BPW // _CHUNK


@functools.cache
def _gather_kernel():
  mesh = plsc.VectorSubcoreMesh(core_axis_name="c", subcore_axis_name="s")

  @functools.partial(
      pl.kernel,
      mesh=mesh,
      out_type=(
          jax.ShapeDtypeStruct((_BATCH, _DIM), jnp.float32),
          jax.ShapeDtypeStruct((_BATCH,), jnp.float32),
          jax.ShapeDtypeStruct((_NEG, _DIM), jnp.float32),
          jax.ShapeDtypeStruct((_NEG,), jnp.float32),
      ),
      scratch_types=(
          pltpu.VMEM((_NCHUNK, _CHUNK), jnp.int32),
          pltpu.VMEM((_BPW, _DIM), jnp.float32),
          pltpu.VMEM((_BPW,), jnp.float32),
          pltpu.VMEM((_SPW,), jnp.int32),
          pltpu.VMEM((_SPW, _DIM), jnp.float32),
          pltpu.VMEM((_SPW,), jnp.float32),
          pltpu.SemaphoreType.DMA,
      ),
      compiler_params=pltpu.CompilerParams(use_tc_tiling_on_sc=False),
  )
  def gather(table_hbm, bias_hbm, tgt_hbm, smp_hbm,
             tw_out, tb_out, sw_out, sb_out,
             idx_v, rows_v, bvals_v, sidx_v, srows_v, sbvals_v, sem):
    wid = lax.axis_index("s") * _NC + lax.axis_index("c")
    base = wid * _BPW
    pltpu.sync_copy(tgt_hbm.at[wid], idx_v)
    pltpu.sync_copy(smp_hbm.at[wid], sidx_v)
    copies = []
    for j in range(_NCHUNK):
      copies.append(pltpu.async_copy(
          table_hbm.at[idx_v.at[j]], rows_v.at[pl.ds(j * _CHUNK, _CHUNK)],
          sem))
      copies.append(pltpu.async_copy(
          bias_hbm.at[idx_v.at[j]], bvals_v.at[pl.ds(j * _CHUNK, _CHUNK)],
          sem))
    copies.append(pltpu.async_copy(table_hbm.at[sidx_v], srows_v, sem))
    copies.append(pltpu.async_copy(bias_hbm.at[sidx_v], sbvals_v, sem))
    for cp in copies:
      cp.wait()
    sbase = wid * _SPW
    pltpu.sync_copy(rows_v, tw_out.at[pl.ds(base, _BPW)])
    pltpu.sync_copy(bvals_v, tb_out.at[pl.ds(base, _BPW)])
    pltpu.sync_copy(srows_v, sw_out.at[pl.ds(sbase, _SPW)])
    pltpu.sync_copy(sbvals_v, sb_out.at[pl.ds(sbase, _SPW)])

  return gather


_BB = 512
_NB = _BATCH // _BB


def _loss_body(logits_ref, tw_ref, tb_ref, tgt_ref, swt_ref, sb_ref, smp_ref,
               out_ref):
  i = pl.program_id(0)
  logits = logits_ref[...]
  tw = tw_ref[...]
  tb = tb_ref[...]
  tgt = tgt_ref[...]
  swt = swt_ref[...]
  sb = sb_ref[...]
  smp = smp_ref[...]

  log_range = jnp.log(jnp.float32(_UNITS) + 1.0)
  nf = jnp.float32(_NEG)

  tf_ = tgt.astype(jnp.float32)
  p_t = jnp.log((tf_ + 2.0) / (tf_ + 1.0)) / log_range
  log_true_ec = jnp.log(1.0 - jnp.exp(nf * jnp.log(1.0 - p_t)))

  sf_ = smp.astype(jnp.float32)
  p_s = jnp.log((sf_ + 2.0) / (sf_ + 1.0)) / log_range
  log_samp_ec = jnp.log(1.0 - jnp.exp(nf * jnp.log(1.0 - p_s)))

  true_logits = (jnp.sum(logits * tw, axis=1, keepdims=True)
                 + tb - log_true_ec)

  samp = lax.dot_general(logits, swt, (((1,), (0,)), ((), ())),
                         preferred_element_type=jnp.float32)
  samp = samp + sb - log_samp_ec
  samp = jnp.where(smp == tgt, samp - 1e9, samp)

  m = jnp.maximum(jnp.max(samp, axis=1, keepdims=True), true_logits)
  s = (jnp.sum(jnp.exp(samp - m), axis=1, keepdims=True)
       + jnp.exp(true_logits - m))
  per_ex = m + jnp.log(s) - true_logits
  blk_sum = jnp.sum(per_ex)

  @pl.when(i == 0)
  def _():
    out_ref[...] = jnp.zeros_like(out_ref)

  out_ref[...] += jnp.reshape(blk_sum, (1, 1))


@functools.cache
def _loss_call():
  return pl.pallas_call(
      _loss_body,
      grid=(_NB,),
      in_specs=[
          pl.BlockSpec((_BB, _DIM), lambda i: (i, 0)),
          pl.BlockSpec((_BB, _DIM), lambda i: (i, 0)),
          pl.BlockSpec((_BB, 1), lambda i: (i, 0)),
          pl.BlockSpec((_BB, 1), lambda i: (i, 0)),
          pl.BlockSpec((_DIM, _NEG), lambda i: (0, 0)),
          pl.BlockSpec((1, _NEG), lambda i: (0, 0)),
          pl.BlockSpec((1, _NEG), lambda i: (0, 0)),
      ],
      out_specs=pl.BlockSpec((1, 1), lambda i: (0, 0)),
      out_shape=jax.ShapeDtypeStruct((1, 1), jnp.float32),
  )


def kernel(logits, targets, kernel, bias):
  table = kernel
  skey = jax.random.fold_in(jax.random.key(42), 7)
  u = jax.random.uniform(skey, (_NEG,), dtype=jnp.float32)
  sampled = jnp.floor(jnp.exp(u * jnp.log(float(_UNITS) + 1.0)))
  sampled = jnp.clip(sampled.astype(jnp.int32) - 1, 0, _UNITS - 1)

  tw, tb, sw, sb = _gather_kernel()(
      table, bias,
      targets.reshape(_NW, _NCHUNK, _CHUNK),
      sampled.reshape(_NW, _SPW))

  loss_sum = _loss_call()(
      logits, tw,
      tb.reshape(_BATCH, 1),
      targets.reshape(_BATCH, 1),
      sw.T,
      sb.reshape(1, _NEG),
      sampled.reshape(1, _NEG))
  return loss_sum[0, 0] / jnp.float32(_BATCH)
